# baseline (speedup 1.0000x reference)
import functools

import jax
import jax.numpy as jnp
from jax import lax
from jax.experimental import pallas as pl
from jax.experimental.pallas import tpu as pltpu
from jax.experimental.pallas import tpu_sc as plsc

_N = 100000
_OBS_D = 17
_ACT_D = 6
_NW = 32
_ROWS = 3136


def _sc_copy_body(obs_hbm, act_hbm, obs_out, act_out, sem_obs, sem_act):
    c = lax.axis_index("c")
    s = lax.axis_index("s")
    wid = s * 2 + c
    base = jnp.minimum(wid * _ROWS, _N - _ROWS)

    c_obs = pltpu.async_copy(
        obs_hbm.at[pl.ds(base, _ROWS), :],
        obs_out.at[pl.ds(base, _ROWS), :], sem_obs)
    c_act = pltpu.async_copy(
        act_hbm.at[pl.ds(base, _ROWS), :],
        act_out.at[pl.ds(base, _ROWS), :], sem_act)
    c_obs.wait()
    c_act.wait()


def kernel(obs_table, act_table):
    k = functools.partial(
        pl.kernel,
        out_type=(
            jax.ShapeDtypeStruct((_N, _OBS_D), jnp.float32),
            jax.ShapeDtypeStruct((_N, _ACT_D), jnp.float32),
        ),
        mesh=plsc.VectorSubcoreMesh(core_axis_name="c", subcore_axis_name="s"),
        scratch_types=[
            pltpu.SemaphoreType.DMA,
            pltpu.SemaphoreType.DMA,
        ],
    )(_sc_copy_body)
    return k(obs_table, act_table)


# speedup vs baseline: 16.1268x; 16.1268x over previous
import functools

import jax
import jax.numpy as jnp
from jax import lax
from jax.experimental import pallas as pl
from jax.experimental.pallas import tpu as pltpu
from jax.experimental.pallas import tpu_sc as plsc

_N = 100000
_OBS_D = 17
_ACT_D = 6
_NW = 32
_ROWS = 3136
_CHUNK = 224
_NCHUNK = _ROWS // _CHUNK


def _sc_copy_body(obs_hbm, act_hbm, obs_out, act_out,
                  obs_v, act_v, sem_in, sem_out):
    c = lax.axis_index("c")
    s = lax.axis_index("s")
    wid = s * 2 + c
    base = jnp.minimum(wid * _ROWS, _N - _ROWS)

    outs = [None] * _NCHUNK
    for i in range(_NCHUNK):
        b = i % 2
        if i >= 2:
            for cp in outs[i - 2]:
                cp.wait()
        lo = base + i * _CHUNK
        in_o = pltpu.async_copy(
            obs_hbm.at[pl.ds(lo, _CHUNK), :], obs_v.at[b], sem_in)
        in_a = pltpu.async_copy(
            act_hbm.at[pl.ds(lo, _CHUNK), :], act_v.at[b], sem_in)
        in_o.wait()
        in_a.wait()
        outs[i] = (
            pltpu.async_copy(
                obs_v.at[b], obs_out.at[pl.ds(lo, _CHUNK), :], sem_out),
            pltpu.async_copy(
                act_v.at[b], act_out.at[pl.ds(lo, _CHUNK), :], sem_out),
        )
    for i in (_NCHUNK - 2, _NCHUNK - 1):
        for cp in outs[i]:
            cp.wait()


def kernel(obs_table, act_table):
    k = functools.partial(
        pl.kernel,
        out_type=(
            jax.ShapeDtypeStruct((_N, _OBS_D), jnp.float32),
            jax.ShapeDtypeStruct((_N, _ACT_D), jnp.float32),
        ),
        mesh=plsc.VectorSubcoreMesh(core_axis_name="c", subcore_axis_name="s"),
        scratch_types=[
            pltpu.VMEM((2, _CHUNK, _OBS_D), jnp.float32),
            pltpu.VMEM((2, _CHUNK, _ACT_D), jnp.float32),
            pltpu.SemaphoreType.DMA,
            pltpu.SemaphoreType.DMA,
        ],
    )(_sc_copy_body)
    return k(obs_table, act_table)
